# Initial kernel scaffold; baseline (speedup 1.0000x reference)
import jax
import jax.numpy as jnp
from jax.experimental import pallas as pl


def kernel(x, coords, W, b, gamma, beta):
    raise NotImplementedError("write your pallas kernel here")


# speedup vs baseline: 17.5239x; 17.5239x over previous
import functools

import jax
import jax.numpy as jnp
from jax import lax
from jax.experimental import pallas as pl
from jax.experimental.pallas import tpu as pltpu
from jax.experimental.pallas import tpu_sc as plsc

_K = 16
_M = 2048
_CIN = 128
_COUT = 256
_EPS = 1e-5
_B = 4
_N = 8192
_S = _B * _M * _K


def _fps_body(px_ref, py_ref, pz_ref, bufx_ref, bufy_ref, bufz_ref):
    px = px_ref[...]
    py = py_ref[...]
    pz = pz_ref[...]
    n_idx = (lax.broadcasted_iota(jnp.int32, (_B, 8, 1024), 1) * 1024
             + lax.broadcasted_iota(jnp.int32, (_B, 8, 1024), 2))

    def body(i, carry):
        dists, far = carry
        sel = n_idx == far
        cx = jnp.sum(jnp.where(sel, px, 0.0), axis=(1, 2), keepdims=True)
        cy = jnp.sum(jnp.where(sel, py, 0.0), axis=(1, 2), keepdims=True)
        cz = jnp.sum(jnp.where(sel, pz, 0.0), axis=(1, 2), keepdims=True)
        bufx_ref[pl.ds(i, 1), :] = cx.reshape(1, _B)
        bufy_ref[pl.ds(i, 1), :] = cy.reshape(1, _B)
        bufz_ref[pl.ds(i, 1), :] = cz.reshape(1, _B)
        dx = px - cx
        dy = py - cy
        dz = pz - cz
        d = (dx * dx + dy * dy) + dz * dz
        dists = jnp.minimum(dists, d)
        mx = jnp.max(dists, axis=(1, 2), keepdims=True)
        nf = jnp.min(jnp.where(dists == mx, n_idx, _N), axis=(1, 2),
                     keepdims=True)
        return dists, nf

    init = (jnp.full((_B, 8, 1024), 1e10, jnp.float32),
            jnp.zeros((_B, 1, 1), jnp.int32))
    lax.fori_loop(0, _M, body, init)


def _run_fps(coords):
    cr = coords.reshape(_B, 3, 8, 1024)
    out_sd = jax.ShapeDtypeStruct((_M, _B), jnp.float32)
    bufx, bufy, bufz = pl.pallas_call(
        _fps_body,
        out_shape=[out_sd, out_sd, out_sd],
    )(cr[:, 0], cr[:, 1], cr[:, 2])
    return bufx, bufy, bufz


_QB = 128


def _knn_body(fps_ref, pt_ref, knn_ref):
    cq = fps_ref[0]
    pt = pt_ref[0]
    cp = lax.dot_general(cq, pt, (((1,), (1,)), ((), ())),
                         preferred_element_type=jnp.float32)
    c2 = jnp.sum(cq * cq, axis=1, keepdims=True)
    p2 = jnp.sum(pt * pt, axis=1, keepdims=True)
    d = (c2 - 2.0 * cp) + jnp.transpose(p2)
    iota_n = lax.broadcasted_iota(jnp.int32, (_QB, _N), 1)
    cols = []
    for _ in range(_K):
        m = jnp.min(d, axis=1, keepdims=True)
        ik = jnp.min(jnp.where(d == m, iota_n, _N), axis=1, keepdims=True)
        cols.append(ik)
        d = jnp.where(iota_n == ik, jnp.inf, d)
    knn_ref[0] = jnp.concatenate(cols, axis=1)


def _run_knn(fps_t, coords_t):
    return pl.pallas_call(
        _knn_body,
        grid=(_B, _M // _QB),
        in_specs=[
            pl.BlockSpec((1, _QB, 3), lambda b, q: (b, q, 0)),
            pl.BlockSpec((1, _N, 3), lambda b, q: (b, 0, 0)),
        ],
        out_specs=pl.BlockSpec((1, _QB, _K), lambda b, q: (b, q, 0)),
        out_shape=jax.ShapeDtypeStruct((_B, _M, _K), jnp.int32),
    )(fps_t, coords_t)


_NB = 512


def _mm_body(x_ref, w_ref, b_ref, zt_ref):
    xb = x_ref[0]
    zt = lax.dot_general(xb, w_ref[...], (((0,), (1,)), ((), ())),
                         preferred_element_type=jnp.float32)
    zt_ref[0] = zt + b_ref[...][None, :]


def _run_mm(x, W, b):
    return pl.pallas_call(
        _mm_body,
        grid=(_B, _N // _NB),
        in_specs=[
            pl.BlockSpec((1, _CIN, _NB), lambda bb, nb: (bb, 0, nb)),
            pl.BlockSpec((_COUT, _CIN), lambda bb, nb: (0, 0)),
            pl.BlockSpec((_COUT,), lambda bb, nb: (0,)),
        ],
        out_specs=pl.BlockSpec((1, _NB, _COUT), lambda bb, nb: (bb, nb, 0)),
        out_shape=jax.ShapeDtypeStruct((_B, _N, _COUT), jnp.float32),
    )(x, W, b)


_NC = 2
_NS = 16
_NW = _NC * _NS


def _make_sc_gather():
    nc, ns = _NC, _NS
    nw = nc * ns
    pairs = _B * _M
    ppt = pairs // nw
    mesh = plsc.VectorSubcoreMesh(core_axis_name="c", subcore_axis_name="s")
    nch = _COUT // 16

    @functools.partial(
        pl.kernel,
        mesh=mesh,
        out_type=[
            jax.ShapeDtypeStruct((pairs, _COUT), jnp.float32),
            jax.ShapeDtypeStruct((pairs, _COUT), jnp.float32),
            jax.ShapeDtypeStruct((nw, _COUT), jnp.float32),
            jax.ShapeDtypeStruct((nw, _COUT), jnp.float32),
        ],
        scratch_types=[
            pltpu.VMEM((ppt, _K), jnp.int32),
            pltpu.VMEM((_K, _COUT), jnp.float32),
            pltpu.VMEM((_COUT,), jnp.float32),
            pltpu.VMEM((_COUT,), jnp.float32),
            pltpu.VMEM((_COUT,), jnp.float32),
            pltpu.VMEM((_COUT,), jnp.float32),
            pltpu.SemaphoreType.DMA,
        ],
    )
    def sc_gather(zt_hbm, gidx_hbm, hmax_hbm, hmin_hbm, s1_hbm, s2_hbm,
                  idx_v, rows_v, maxb, minb, s1v, s2v, sem):
        wid = lax.axis_index("s") * nc + lax.axis_index("c")
        base = wid * ppt
        pltpu.sync_copy(gidx_hbm.at[pl.ds(base, ppt)], idx_v)
        zero = jnp.zeros((16,), jnp.float32)
        for c in range(nch):
            s1v[pl.ds(16 * c, 16)] = zero
            s2v[pl.ds(16 * c, 16)] = zero

        def pair_body(j, carry):
            cp = pltpu.async_copy(zt_hbm.at[idx_v.at[j]], rows_v, sem)
            cp.wait()
            for c in range(nch):
                sl = pl.ds(16 * c, 16)
                vmax = rows_v[0, sl]
                vmin = vmax
                vsum = vmax
                vsq = vmax * vmax
                for r in range(1, _K):
                    v = rows_v[r, sl]
                    vmax = jnp.maximum(vmax, v)
                    vmin = jnp.minimum(vmin, v)
                    vsum = vsum + v
                    vsq = vsq + v * v
                maxb[sl] = vmax
                minb[sl] = vmin
                s1v[sl] = s1v[sl] + vsum
                s2v[sl] = s2v[sl] + vsq
            pltpu.sync_copy(maxb, hmax_hbm.at[base + j])
            pltpu.sync_copy(minb, hmin_hbm.at[base + j])
            return carry

        lax.fori_loop(0, ppt, pair_body, 0)
        pltpu.sync_copy(s1v, s1_hbm.at[wid])
        pltpu.sync_copy(s2v, s2_hbm.at[wid])

    return sc_gather


_sc_gather_cache = []


def _gather_pool(zt_flat, gidx):
    if not _sc_gather_cache:
        _sc_gather_cache.append(_make_sc_gather())
    return _sc_gather_cache[0](zt_flat, gidx)


_FB = 1024


def _fin_body(hmax_ref, hmin_ref, s1_ref, s2_ref, g_ref, bt_ref, y_ref):
    s1 = jnp.sum(s1_ref[...], axis=0, keepdims=True)
    s2 = jnp.sum(s2_ref[...], axis=0, keepdims=True)
    mean = s1 / _S
    var = s2 / _S - mean * mean
    inv = g_ref[...][None, :] / jnp.sqrt(var + _EPS)
    shift = bt_ref[...][None, :] - mean * inv
    h = jnp.where(inv >= 0.0, hmax_ref[...], hmin_ref[...])
    y_ref[...] = jnp.maximum(h * inv + shift, 0.0)


def _run_finalize(hmax, hmin, s1p, s2p, gamma, beta):
    nw = s1p.shape[0]
    pairs = hmax.shape[0]
    return pl.pallas_call(
        _fin_body,
        grid=(pairs // _FB,),
        in_specs=[
            pl.BlockSpec((_FB, _COUT), lambda i: (i, 0)),
            pl.BlockSpec((_FB, _COUT), lambda i: (i, 0)),
            pl.BlockSpec((nw, _COUT), lambda i: (0, 0)),
            pl.BlockSpec((nw, _COUT), lambda i: (0, 0)),
            pl.BlockSpec((_COUT,), lambda i: (0,)),
            pl.BlockSpec((_COUT,), lambda i: (0,)),
        ],
        out_specs=pl.BlockSpec((_FB, _COUT), lambda i: (i, 0)),
        out_shape=jax.ShapeDtypeStruct((pairs, _COUT), jnp.float32),
    )(hmax, hmin, s1p, s2p, gamma, beta)


def kernel(x, coords, W, b, gamma, beta):
    bufx, bufy, bufz = _run_fps(coords)
    fps_t = jnp.transpose(jnp.stack([bufx, bufy, bufz], axis=-1),
                          (1, 0, 2))
    fps_coords = jnp.transpose(fps_t, (0, 2, 1))
    knn = _run_knn(fps_t, jnp.transpose(coords, (0, 2, 1)))
    zt = _run_mm(x, W, b)
    gidx = (knn + (jnp.arange(_B, dtype=jnp.int32) * _N)[:, None, None])
    gidx = gidx.reshape(_B * _M, _K)
    hmax, hmin, s1p, s2p = _gather_pool(zt.reshape(_B * _N, _COUT), gidx)
    y_t = _run_finalize(hmax, hmin, s1p, s2p, gamma, beta)
    y = jnp.transpose(y_t.reshape(_B, _M, _COUT), (0, 2, 1))
    return (y, fps_coords)


# speedup vs baseline: 20.6914x; 1.1808x over previous
import functools

import jax
import jax.numpy as jnp
from jax import lax
from jax.experimental import pallas as pl
from jax.experimental.pallas import tpu as pltpu
from jax.experimental.pallas import tpu_sc as plsc

_K = 16
_M = 2048
_CIN = 128
_COUT = 256
_EPS = 1e-5
_B = 4
_N = 8192
_S = _B * _M * _K


def _fps_body(px_ref, py_ref, pz_ref, bufx_ref, bufy_ref, bufz_ref):
    px = px_ref[...]
    py = py_ref[...]
    pz = pz_ref[...]
    n_idx = (lax.broadcasted_iota(jnp.int32, (_B, 8, 1024), 1) * 1024
             + lax.broadcasted_iota(jnp.int32, (_B, 8, 1024), 2)
             ).astype(jnp.float32)

    def body(i, carry):
        dists, far = carry
        sel = n_idx == far
        cx = jnp.sum(jnp.where(sel, px, 0.0), axis=(1, 2), keepdims=True)
        cy = jnp.sum(jnp.where(sel, py, 0.0), axis=(1, 2), keepdims=True)
        cz = jnp.sum(jnp.where(sel, pz, 0.0), axis=(1, 2), keepdims=True)
        bufx_ref[pl.ds(i, 1), :] = cx.reshape(1, _B)
        bufy_ref[pl.ds(i, 1), :] = cy.reshape(1, _B)
        bufz_ref[pl.ds(i, 1), :] = cz.reshape(1, _B)
        dx = px - cx
        dy = py - cy
        dz = pz - cz
        d = (dx * dx + dy * dy) + dz * dz
        dists = jnp.minimum(dists, d)
        mx = jnp.max(dists, axis=(1, 2), keepdims=True)
        nf = jnp.min(jnp.where(dists == mx, n_idx, float(_N)), axis=(1, 2),
                     keepdims=True)
        return dists, nf

    init = (jnp.full((_B, 8, 1024), 1e10, jnp.float32),
            jnp.zeros((_B, 1, 1), jnp.float32))
    lax.fori_loop(0, _M, body, init)


def _run_fps(coords):
    cr = coords.reshape(_B, 3, 8, 1024)
    out_sd = jax.ShapeDtypeStruct((_M, _B), jnp.float32)
    bufx, bufy, bufz = pl.pallas_call(
        _fps_body,
        out_shape=[out_sd, out_sd, out_sd],
    )(cr[:, 0], cr[:, 1], cr[:, 2])
    return bufx, bufy, bufz


_QB = 128


def _knn_body(fps_ref, pt_ref, knn_ref):
    cq = fps_ref[0]
    pt = pt_ref[0]
    cp = lax.dot_general(cq, pt, (((1,), (1,)), ((), ())),
                         preferred_element_type=jnp.float32)
    c2 = jnp.sum(cq * cq, axis=1, keepdims=True)
    p2 = jnp.sum(pt * pt, axis=1, keepdims=True)
    d = (c2 - 2.0 * cp) + jnp.transpose(p2)
    iota_n = lax.broadcasted_iota(jnp.int32, (_QB, _N), 1).astype(jnp.float32)
    cols = []
    for _ in range(_K):
        m = jnp.min(d, axis=1, keepdims=True)
        ik = jnp.min(jnp.where(d == m, iota_n, float(_N)), axis=1,
                     keepdims=True)
        cols.append(ik)
        d = jnp.where(iota_n == ik, jnp.inf, d)
    knn_ref[0] = jnp.concatenate(cols, axis=1).astype(jnp.int32)


def _run_knn(fps_t, coords_t):
    return pl.pallas_call(
        _knn_body,
        grid=(_B, _M // _QB),
        in_specs=[
            pl.BlockSpec((1, _QB, 3), lambda b, q: (b, q, 0)),
            pl.BlockSpec((1, _N, 3), lambda b, q: (b, 0, 0)),
        ],
        out_specs=pl.BlockSpec((1, _QB, _K), lambda b, q: (b, q, 0)),
        out_shape=jax.ShapeDtypeStruct((_B, _M, _K), jnp.int32),
    )(fps_t, coords_t)


_NB = 512


def _mm_body(x_ref, w_ref, b_ref, zt_ref):
    xb = x_ref[0]
    zt = lax.dot_general(xb, w_ref[...], (((0,), (1,)), ((), ())),
                         preferred_element_type=jnp.float32)
    zt_ref[0] = zt + b_ref[...][None, :]


def _run_mm(x, W, b):
    return pl.pallas_call(
        _mm_body,
        grid=(_B, _N // _NB),
        in_specs=[
            pl.BlockSpec((1, _CIN, _NB), lambda bb, nb: (bb, 0, nb)),
            pl.BlockSpec((_COUT, _CIN), lambda bb, nb: (0, 0)),
            pl.BlockSpec((_COUT,), lambda bb, nb: (0,)),
        ],
        out_specs=pl.BlockSpec((1, _NB, _COUT), lambda bb, nb: (bb, nb, 0)),
        out_shape=jax.ShapeDtypeStruct((_B, _N, _COUT), jnp.float32),
    )(x, W, b)


_NC = 2
_NS = 16
_NW = _NC * _NS


def _make_sc_gather():
    nc, ns = _NC, _NS
    nw = nc * ns
    pairs = _B * _M
    ppt = pairs // nw
    mesh = plsc.VectorSubcoreMesh(core_axis_name="c", subcore_axis_name="s")
    nch = _COUT // 16

    @functools.partial(
        pl.kernel,
        mesh=mesh,
        out_type=[
            jax.ShapeDtypeStruct((pairs, _COUT), jnp.float32),
            jax.ShapeDtypeStruct((pairs, _COUT), jnp.float32),
            jax.ShapeDtypeStruct((nw, _COUT), jnp.float32),
            jax.ShapeDtypeStruct((nw, _COUT), jnp.float32),
        ],
        scratch_types=[
            pltpu.VMEM((ppt, _K), jnp.int32),
            pltpu.VMEM((_K, _COUT), jnp.float32),
            pltpu.VMEM((_COUT,), jnp.float32),
            pltpu.VMEM((_COUT,), jnp.float32),
            pltpu.VMEM((_COUT,), jnp.float32),
            pltpu.VMEM((_COUT,), jnp.float32),
            pltpu.SemaphoreType.DMA,
        ],
    )
    def sc_gather(zt_hbm, gidx_hbm, hmax_hbm, hmin_hbm, s1_hbm, s2_hbm,
                  idx_v, rows_v, maxb, minb, s1v, s2v, sem):
        wid = lax.axis_index("s") * nc + lax.axis_index("c")
        base = wid * ppt
        pltpu.sync_copy(gidx_hbm.at[pl.ds(base, ppt)], idx_v)
        zero = jnp.zeros((16,), jnp.float32)
        for c in range(nch):
            s1v[pl.ds(16 * c, 16)] = zero
            s2v[pl.ds(16 * c, 16)] = zero

        def pair_body(j, carry):
            cp = pltpu.async_copy(zt_hbm.at[idx_v.at[j]], rows_v, sem)
            cp.wait()
            for c in range(nch):
                sl = pl.ds(16 * c, 16)
                vmax = rows_v[0, sl]
                vmin = vmax
                vsum = vmax
                vsq = vmax * vmax
                for r in range(1, _K):
                    v = rows_v[r, sl]
                    vmax = jnp.maximum(vmax, v)
                    vmin = jnp.minimum(vmin, v)
                    vsum = vsum + v
                    vsq = vsq + v * v
                maxb[sl] = vmax
                minb[sl] = vmin
                s1v[sl] = s1v[sl] + vsum
                s2v[sl] = s2v[sl] + vsq
            pltpu.sync_copy(maxb, hmax_hbm.at[base + j])
            pltpu.sync_copy(minb, hmin_hbm.at[base + j])
            return carry

        lax.fori_loop(0, ppt, pair_body, 0)
        pltpu.sync_copy(s1v, s1_hbm.at[wid])
        pltpu.sync_copy(s2v, s2_hbm.at[wid])

    return sc_gather


_sc_gather_cache = []


def _gather_pool(zt_flat, gidx):
    if not _sc_gather_cache:
        _sc_gather_cache.append(_make_sc_gather())
    return _sc_gather_cache[0](zt_flat, gidx)


_FB = 1024


def _fin_body(hmax_ref, hmin_ref, s1_ref, s2_ref, g_ref, bt_ref, y_ref):
    s1 = jnp.sum(s1_ref[...], axis=0, keepdims=True)
    s2 = jnp.sum(s2_ref[...], axis=0, keepdims=True)
    mean = s1 / _S
    var = s2 / _S - mean * mean
    inv = g_ref[...][None, :] / jnp.sqrt(var + _EPS)
    shift = bt_ref[...][None, :] - mean * inv
    h = jnp.where(inv >= 0.0, hmax_ref[...], hmin_ref[...])
    y_ref[...] = jnp.maximum(h * inv + shift, 0.0)


def _run_finalize(hmax, hmin, s1p, s2p, gamma, beta):
    nw = s1p.shape[0]
    pairs = hmax.shape[0]
    return pl.pallas_call(
        _fin_body,
        grid=(pairs // _FB,),
        in_specs=[
            pl.BlockSpec((_FB, _COUT), lambda i: (i, 0)),
            pl.BlockSpec((_FB, _COUT), lambda i: (i, 0)),
            pl.BlockSpec((nw, _COUT), lambda i: (0, 0)),
            pl.BlockSpec((nw, _COUT), lambda i: (0, 0)),
            pl.BlockSpec((_COUT,), lambda i: (0,)),
            pl.BlockSpec((_COUT,), lambda i: (0,)),
        ],
        out_specs=pl.BlockSpec((_FB, _COUT), lambda i: (i, 0)),
        out_shape=jax.ShapeDtypeStruct((pairs, _COUT), jnp.float32),
    )(hmax, hmin, s1p, s2p, gamma, beta)


def kernel(x, coords, W, b, gamma, beta):
    bufx, bufy, bufz = _run_fps(coords)
    fps_t = jnp.transpose(jnp.stack([bufx, bufy, bufz], axis=-1),
                          (1, 0, 2))
    fps_coords = jnp.transpose(fps_t, (0, 2, 1))
    knn = _run_knn(fps_t, jnp.transpose(coords, (0, 2, 1)))
    zt = _run_mm(x, W, b)
    gidx = (knn + (jnp.arange(_B, dtype=jnp.int32) * _N)[:, None, None])
    gidx = gidx.reshape(_B * _M, _K)
    hmax, hmin, s1p, s2p = _gather_pool(zt.reshape(_B * _N, _COUT), gidx)
    y_t = _run_finalize(hmax, hmin, s1p, s2p, gamma, beta)
    y = jnp.transpose(y_t.reshape(_B, _M, _COUT), (0, 2, 1))
    return (y, fps_coords)


# speedup vs baseline: 22.0563x; 1.0660x over previous
import functools

import jax
import jax.numpy as jnp
from jax import lax
from jax.experimental import pallas as pl
from jax.experimental.pallas import tpu as pltpu
from jax.experimental.pallas import tpu_sc as plsc

_K = 16
_M = 2048
_CIN = 128
_COUT = 256
_EPS = 1e-5
_B = 4
_N = 8192
_S = _B * _M * _K


def _fps_body(px_ref, py_ref, pz_ref, bufx_ref, bufy_ref, bufz_ref):
    px = px_ref[...]
    py = py_ref[...]
    pz = pz_ref[...]
    n_idx = (lax.broadcasted_iota(jnp.int32, (_B, 8, 1024), 1) * 1024
             + lax.broadcasted_iota(jnp.int32, (_B, 8, 1024), 2)
             ).astype(jnp.float32)

    def body(i, carry):
        dists, far = carry
        sel = n_idx == far
        cx = jnp.sum(jnp.where(sel, px, 0.0), axis=(1, 2), keepdims=True)
        cy = jnp.sum(jnp.where(sel, py, 0.0), axis=(1, 2), keepdims=True)
        cz = jnp.sum(jnp.where(sel, pz, 0.0), axis=(1, 2), keepdims=True)
        bufx_ref[pl.ds(i, 1), :] = cx.reshape(1, _B)
        bufy_ref[pl.ds(i, 1), :] = cy.reshape(1, _B)
        bufz_ref[pl.ds(i, 1), :] = cz.reshape(1, _B)
        dx = px - cx
        dy = py - cy
        dz = pz - cz
        d = (dx * dx + dy * dy) + dz * dz
        dists = jnp.minimum(dists, d)
        mx = jnp.max(dists, axis=(1, 2), keepdims=True)
        nf = jnp.min(jnp.where(dists == mx, n_idx, float(_N)), axis=(1, 2),
                     keepdims=True)
        return dists, nf

    init = (jnp.full((_B, 8, 1024), 1e10, jnp.float32),
            jnp.zeros((_B, 1, 1), jnp.float32))
    lax.fori_loop(0, _M, body, init)


def _run_fps(coords):
    cr = coords.reshape(_B, 3, 8, 1024)
    out_sd = jax.ShapeDtypeStruct((_M, _B), jnp.float32)
    bufx, bufy, bufz = pl.pallas_call(
        _fps_body,
        out_shape=[out_sd, out_sd, out_sd],
    )(cr[:, 0], cr[:, 1], cr[:, 2])
    return bufx, bufy, bufz


_QB = 128


def _knn_body(fps_ref, pt_ref, knn_ref):
    cq = fps_ref[0]
    pt = pt_ref[0]
    cp = lax.dot_general(cq, pt, (((1,), (1,)), ((), ())),
                         preferred_element_type=jnp.float32)
    c2 = jnp.sum(cq * cq, axis=1, keepdims=True)
    p2 = jnp.sum(pt * pt, axis=1, keepdims=True)
    d = (c2 - 2.0 * cp) + jnp.transpose(p2)
    iota_n = lax.broadcasted_iota(jnp.int32, (_QB, _N), 1).astype(jnp.float32)
    cols = []
    for _ in range(_K):
        m = jnp.min(d, axis=1, keepdims=True)
        ik = jnp.min(jnp.where(d == m, iota_n, float(_N)), axis=1,
                     keepdims=True)
        cols.append(ik)
        d = jnp.where(iota_n == ik, jnp.inf, d)
    knn_ref[0] = jnp.concatenate(cols, axis=1).astype(jnp.int32)


def _run_knn(fps_t, coords_t):
    return pl.pallas_call(
        _knn_body,
        grid=(_B, _M // _QB),
        in_specs=[
            pl.BlockSpec((1, _QB, 3), lambda b, q: (b, q, 0)),
            pl.BlockSpec((1, _N, 3), lambda b, q: (b, 0, 0)),
        ],
        out_specs=pl.BlockSpec((1, _QB, _K), lambda b, q: (b, q, 0)),
        out_shape=jax.ShapeDtypeStruct((_B, _M, _K), jnp.int32),
    )(fps_t, coords_t)


_NB = 512


def _mm_body(x_ref, w_ref, b_ref, zt_ref):
    xb = x_ref[0]
    zt = lax.dot_general(xb, w_ref[...], (((0,), (1,)), ((), ())),
                         preferred_element_type=jnp.float32)
    zt_ref[0] = zt + b_ref[...][None, :]


def _run_mm(x, W, b):
    return pl.pallas_call(
        _mm_body,
        grid=(_B, _N // _NB),
        in_specs=[
            pl.BlockSpec((1, _CIN, _NB), lambda bb, nb: (bb, 0, nb)),
            pl.BlockSpec((_COUT, _CIN), lambda bb, nb: (0, 0)),
            pl.BlockSpec((_COUT,), lambda bb, nb: (0,)),
        ],
        out_specs=pl.BlockSpec((1, _NB, _COUT), lambda bb, nb: (bb, nb, 0)),
        out_shape=jax.ShapeDtypeStruct((_B, _N, _COUT), jnp.float32),
    )(x, W, b)


_NC = 2
_NS = 16
_NW = _NC * _NS


_PPG = 8


def _make_sc_gather():
    nc, ns = _NC, _NS
    nw = nc * ns
    pairs = _B * _M
    ppt = pairs // nw
    ng = ppt // _PPG
    mesh = plsc.VectorSubcoreMesh(core_axis_name="c", subcore_axis_name="s")
    nch = _COUT // 16

    @functools.partial(
        pl.kernel,
        mesh=mesh,
        out_type=[
            jax.ShapeDtypeStruct((pairs, _COUT), jnp.float32),
            jax.ShapeDtypeStruct((pairs, _COUT), jnp.float32),
            jax.ShapeDtypeStruct((nw, _COUT), jnp.float32),
            jax.ShapeDtypeStruct((nw, _COUT), jnp.float32),
        ],
        scratch_types=[
            pltpu.VMEM((ppt * _K,), jnp.int32),
            pltpu.VMEM((_PPG * _K, _COUT), jnp.float32),
            pltpu.VMEM((_PPG * _K, _COUT), jnp.float32),
            pltpu.VMEM((_PPG, _COUT), jnp.float32),
            pltpu.VMEM((_PPG, _COUT), jnp.float32),
            pltpu.VMEM((_COUT,), jnp.float32),
            pltpu.VMEM((_COUT,), jnp.float32),
            pltpu.SemaphoreType.DMA,
            pltpu.SemaphoreType.DMA,
        ],
    )
    def sc_gather(zt_hbm, gidx_hbm, hmax_hbm, hmin_hbm, s1_hbm, s2_hbm,
                  idx_v, rows0, rows1, maxb, minb, s1v, s2v, sem0, sem1):
        wid = lax.axis_index("s") * nc + lax.axis_index("c")
        base = wid * ppt
        pltpu.sync_copy(gidx_hbm.at[pl.ds(base * _K, ppt * _K)], idx_v)
        zero = jnp.zeros((16,), jnp.float32)
        for c in range(nch):
            s1v[pl.ds(16 * c, 16)] = zero
            s2v[pl.ds(16 * c, 16)] = zero

        def group_compute(rows, g):
            def pair_body(p, carry):
                r0 = p * _K
                for c in range(nch):
                    sl = pl.ds(16 * c, 16)
                    vmax = rows[r0, sl]
                    vmin = vmax
                    vsum = vmax
                    vsq = vmax * vmax
                    for r in range(1, _K):
                        v = rows[r0 + r, sl]
                        vmax = jnp.maximum(vmax, v)
                        vmin = jnp.minimum(vmin, v)
                        vsum = vsum + v
                        vsq = vsq + v * v
                    maxb[p, sl] = vmax
                    minb[p, sl] = vmin
                    s1v[sl] = s1v[sl] + vsum
                    s2v[sl] = s2v[sl] + vsq
                return carry

            lax.fori_loop(0, _PPG, pair_body, 0)
            pltpu.sync_copy(maxb, hmax_hbm.at[pl.ds(base + g * _PPG, _PPG)])
            pltpu.sync_copy(minb, hmin_hbm.at[pl.ds(base + g * _PPG, _PPG)])

        def two_groups(t, carry):
            g0 = 2 * t
            g1 = 2 * t + 1
            cp0 = pltpu.async_copy(
                zt_hbm.at[idx_v.at[pl.ds(g0 * _PPG * _K, _PPG * _K)]],
                rows0, sem0)
            cp1 = pltpu.async_copy(
                zt_hbm.at[idx_v.at[pl.ds(g1 * _PPG * _K, _PPG * _K)]],
                rows1, sem1)
            cp0.wait()
            group_compute(rows0, g0)
            cp1.wait()
            group_compute(rows1, g1)
            return carry

        lax.fori_loop(0, ng // 2, two_groups, 0)
        pltpu.sync_copy(s1v, s1_hbm.at[wid])
        pltpu.sync_copy(s2v, s2_hbm.at[wid])

    return sc_gather


_sc_gather_cache = []


def _gather_pool(zt_flat, gidx):
    if not _sc_gather_cache:
        _sc_gather_cache.append(_make_sc_gather())
    return _sc_gather_cache[0](zt_flat, gidx)


_FB = 1024


def _fin_body(hmax_ref, hmin_ref, s1_ref, s2_ref, g_ref, bt_ref, y_ref):
    s1 = jnp.sum(s1_ref[...], axis=0, keepdims=True)
    s2 = jnp.sum(s2_ref[...], axis=0, keepdims=True)
    mean = s1 / _S
    var = s2 / _S - mean * mean
    inv = g_ref[...][None, :] / jnp.sqrt(var + _EPS)
    shift = bt_ref[...][None, :] - mean * inv
    h = jnp.where(inv >= 0.0, hmax_ref[...], hmin_ref[...])
    y_ref[...] = jnp.maximum(h * inv + shift, 0.0)


def _run_finalize(hmax, hmin, s1p, s2p, gamma, beta):
    nw = s1p.shape[0]
    pairs = hmax.shape[0]
    return pl.pallas_call(
        _fin_body,
        grid=(pairs // _FB,),
        in_specs=[
            pl.BlockSpec((_FB, _COUT), lambda i: (i, 0)),
            pl.BlockSpec((_FB, _COUT), lambda i: (i, 0)),
            pl.BlockSpec((nw, _COUT), lambda i: (0, 0)),
            pl.BlockSpec((nw, _COUT), lambda i: (0, 0)),
            pl.BlockSpec((_COUT,), lambda i: (0,)),
            pl.BlockSpec((_COUT,), lambda i: (0,)),
        ],
        out_specs=pl.BlockSpec((_FB, _COUT), lambda i: (i, 0)),
        out_shape=jax.ShapeDtypeStruct((pairs, _COUT), jnp.float32),
    )(hmax, hmin, s1p, s2p, gamma, beta)


def kernel(x, coords, W, b, gamma, beta):
    bufx, bufy, bufz = _run_fps(coords)
    fps_t = jnp.transpose(jnp.stack([bufx, bufy, bufz], axis=-1),
                          (1, 0, 2))
    fps_coords = jnp.transpose(fps_t, (0, 2, 1))
    knn = _run_knn(fps_t, jnp.transpose(coords, (0, 2, 1)))
    zt = _run_mm(x, W, b)
    gidx = (knn + (jnp.arange(_B, dtype=jnp.int32) * _N)[:, None, None])
    gidx = gidx.reshape(_B * _M * _K)
    hmax, hmin, s1p, s2p = _gather_pool(zt.reshape(_B * _N, _COUT), gidx)
    y_t = _run_finalize(hmax, hmin, s1p, s2p, gamma, beta)
    y = jnp.transpose(y_t.reshape(_B, _M, _COUT), (0, 2, 1))
    return (y, fps_coords)


# speedup vs baseline: 22.2034x; 1.0067x over previous
import functools

import jax
import jax.numpy as jnp
from jax import lax
from jax.experimental import pallas as pl
from jax.experimental.pallas import tpu as pltpu
from jax.experimental.pallas import tpu_sc as plsc

_K = 16
_M = 2048
_CIN = 128
_COUT = 256
_EPS = 1e-5
_B = 4
_N = 8192
_S = _B * _M * _K


def _fps_body(p4_ref, buf_ref):
    p4 = p4_ref[...]
    px = p4[:, 0]
    py = p4[:, 1]
    pz = p4[:, 2]
    n_idx = (lax.broadcasted_iota(jnp.int32, (_B, 8, 1024), 1) * 1024
             + lax.broadcasted_iota(jnp.int32, (_B, 8, 1024), 2)
             ).astype(jnp.float32)
    n_idx4 = n_idx[:, None]

    def body(i, carry):
        dists, far = carry
        sel4 = n_idx4 == far[:, None]
        c = jnp.sum(jnp.where(sel4, p4, 0.0), axis=(2, 3),
                    keepdims=True)
        buf_ref[pl.ds(i, 1), :] = c.reshape(1, 3 * _B)
        dx = px - c[:, 0]
        dy = py - c[:, 1]
        dz = pz - c[:, 2]
        d = (dx * dx + dy * dy) + dz * dz
        dists = jnp.minimum(dists, d)
        mx = jnp.max(dists, axis=(1, 2), keepdims=True)
        nf = jnp.min(jnp.where(dists == mx, n_idx, float(_N)), axis=(1, 2),
                     keepdims=True)
        return dists, nf

    init = (jnp.full((_B, 8, 1024), 1e10, jnp.float32),
            jnp.zeros((_B, 1, 1), jnp.float32))
    lax.fori_loop(0, _M, body, init)


def _run_fps(coords):
    cr = coords.reshape(_B, 3, 8, 1024)
    buf = pl.pallas_call(
        _fps_body,
        out_shape=jax.ShapeDtypeStruct((_M, 3 * _B), jnp.float32),
    )(cr)
    return buf


_QB = 128


def _knn_body(fps_ref, pt_ref, knn_ref):
    cq = fps_ref[0]
    pt = pt_ref[0]
    cp = lax.dot_general(cq, pt, (((1,), (1,)), ((), ())),
                         preferred_element_type=jnp.float32)
    c2 = jnp.sum(cq * cq, axis=1, keepdims=True)
    p2 = jnp.sum(pt * pt, axis=1, keepdims=True)
    d = (c2 - 2.0 * cp) + jnp.transpose(p2)
    iota_n = lax.broadcasted_iota(jnp.int32, (_QB, _N), 1).astype(jnp.float32)
    cols = []
    for _ in range(_K):
        m = jnp.min(d, axis=1, keepdims=True)
        ik = jnp.min(jnp.where(d == m, iota_n, float(_N)), axis=1,
                     keepdims=True)
        cols.append(ik)
        d = jnp.where(iota_n == ik, jnp.inf, d)
    knn_ref[0] = jnp.concatenate(cols, axis=1).astype(jnp.int32)


def _run_knn(fps_t, coords_t):
    return pl.pallas_call(
        _knn_body,
        grid=(_B, _M // _QB),
        in_specs=[
            pl.BlockSpec((1, _QB, 3), lambda b, q: (b, q, 0)),
            pl.BlockSpec((1, _N, 3), lambda b, q: (b, 0, 0)),
        ],
        out_specs=pl.BlockSpec((1, _QB, _K), lambda b, q: (b, q, 0)),
        out_shape=jax.ShapeDtypeStruct((_B, _M, _K), jnp.int32),
    )(fps_t, coords_t)


_NB = 512


def _mm_body(x_ref, w_ref, b_ref, zt_ref):
    xb = x_ref[0]
    zt = lax.dot_general(xb, w_ref[...], (((0,), (1,)), ((), ())),
                         preferred_element_type=jnp.float32)
    zt_ref[0] = zt + b_ref[...][None, :]


def _run_mm(x, W, b):
    return pl.pallas_call(
        _mm_body,
        grid=(_B, _N // _NB),
        in_specs=[
            pl.BlockSpec((1, _CIN, _NB), lambda bb, nb: (bb, 0, nb)),
            pl.BlockSpec((_COUT, _CIN), lambda bb, nb: (0, 0)),
            pl.BlockSpec((_COUT,), lambda bb, nb: (0,)),
        ],
        out_specs=pl.BlockSpec((1, _NB, _COUT), lambda bb, nb: (bb, nb, 0)),
        out_shape=jax.ShapeDtypeStruct((_B, _N, _COUT), jnp.float32),
    )(x, W, b)


_NC = 2
_NS = 16
_NW = _NC * _NS


_PPG = 8


def _make_sc_gather():
    nc, ns = _NC, _NS
    nw = nc * ns
    pairs = _B * _M
    ppt = pairs // nw
    ng = ppt // _PPG
    mesh = plsc.VectorSubcoreMesh(core_axis_name="c", subcore_axis_name="s")
    nch = _COUT // 16

    @functools.partial(
        pl.kernel,
        mesh=mesh,
        out_type=[
            jax.ShapeDtypeStruct((pairs, _COUT), jnp.float32),
            jax.ShapeDtypeStruct((pairs, _COUT), jnp.float32),
            jax.ShapeDtypeStruct((nw, _COUT), jnp.float32),
            jax.ShapeDtypeStruct((nw, _COUT), jnp.float32),
        ],
        scratch_types=[
            pltpu.VMEM((ppt * _K,), jnp.int32),
            pltpu.VMEM((_PPG * _K, _COUT), jnp.float32),
            pltpu.VMEM((_PPG * _K, _COUT), jnp.float32),
            pltpu.VMEM((_PPG, _COUT), jnp.float32),
            pltpu.VMEM((_PPG, _COUT), jnp.float32),
            pltpu.VMEM((_COUT,), jnp.float32),
            pltpu.VMEM((_COUT,), jnp.float32),
            pltpu.SemaphoreType.DMA,
            pltpu.SemaphoreType.DMA,
        ],
    )
    def sc_gather(zt_hbm, gidx_hbm, hmax_hbm, hmin_hbm, s1_hbm, s2_hbm,
                  idx_v, rows0, rows1, maxb, minb, s1v, s2v, sem0, sem1):
        wid = lax.axis_index("s") * nc + lax.axis_index("c")
        base = wid * ppt
        pltpu.sync_copy(gidx_hbm.at[pl.ds(base * _K, ppt * _K)], idx_v)
        zero = jnp.zeros((16,), jnp.float32)
        for c in range(nch):
            s1v[pl.ds(16 * c, 16)] = zero
            s2v[pl.ds(16 * c, 16)] = zero

        def group_compute(rows, g):
            def pair_body(p, carry):
                r0 = p * _K
                for c in range(nch):
                    sl = pl.ds(16 * c, 16)
                    vmax = rows[r0, sl]
                    vmin = vmax
                    vsum = vmax
                    vsq = vmax * vmax
                    for r in range(1, _K):
                        v = rows[r0 + r, sl]
                        vmax = jnp.maximum(vmax, v)
                        vmin = jnp.minimum(vmin, v)
                        vsum = vsum + v
                        vsq = vsq + v * v
                    maxb[p, sl] = vmax
                    minb[p, sl] = vmin
                    s1v[sl] = s1v[sl] + vsum
                    s2v[sl] = s2v[sl] + vsq
                return carry

            lax.fori_loop(0, _PPG, pair_body, 0)
            pltpu.sync_copy(maxb, hmax_hbm.at[pl.ds(base + g * _PPG, _PPG)])
            pltpu.sync_copy(minb, hmin_hbm.at[pl.ds(base + g * _PPG, _PPG)])

        def two_groups(t, carry):
            g0 = 2 * t
            g1 = 2 * t + 1
            cp0 = pltpu.async_copy(
                zt_hbm.at[idx_v.at[pl.ds(g0 * _PPG * _K, _PPG * _K)]],
                rows0, sem0)
            cp1 = pltpu.async_copy(
                zt_hbm.at[idx_v.at[pl.ds(g1 * _PPG * _K, _PPG * _K)]],
                rows1, sem1)
            cp0.wait()
            group_compute(rows0, g0)
            cp1.wait()
            group_compute(rows1, g1)
            return carry

        lax.fori_loop(0, ng // 2, two_groups, 0)
        pltpu.sync_copy(s1v, s1_hbm.at[wid])
        pltpu.sync_copy(s2v, s2_hbm.at[wid])

    return sc_gather


_sc_gather_cache = []


def _gather_pool(zt_flat, gidx):
    if not _sc_gather_cache:
        _sc_gather_cache.append(_make_sc_gather())
    return _sc_gather_cache[0](zt_flat, gidx)


_FB = 1024


def _fin_body(hmax_ref, hmin_ref, s1_ref, s2_ref, g_ref, bt_ref, y_ref):
    s1 = jnp.sum(s1_ref[...], axis=0, keepdims=True)
    s2 = jnp.sum(s2_ref[...], axis=0, keepdims=True)
    mean = s1 / _S
    var = s2 / _S - mean * mean
    inv = g_ref[...][None, :] / jnp.sqrt(var + _EPS)
    shift = bt_ref[...][None, :] - mean * inv
    h = jnp.where(inv >= 0.0, hmax_ref[...], hmin_ref[...])
    y_ref[...] = jnp.maximum(h * inv + shift, 0.0)


def _run_finalize(hmax, hmin, s1p, s2p, gamma, beta):
    nw = s1p.shape[0]
    pairs = hmax.shape[0]
    return pl.pallas_call(
        _fin_body,
        grid=(pairs // _FB,),
        in_specs=[
            pl.BlockSpec((_FB, _COUT), lambda i: (i, 0)),
            pl.BlockSpec((_FB, _COUT), lambda i: (i, 0)),
            pl.BlockSpec((nw, _COUT), lambda i: (0, 0)),
            pl.BlockSpec((nw, _COUT), lambda i: (0, 0)),
            pl.BlockSpec((_COUT,), lambda i: (0,)),
            pl.BlockSpec((_COUT,), lambda i: (0,)),
        ],
        out_specs=pl.BlockSpec((_FB, _COUT), lambda i: (i, 0)),
        out_shape=jax.ShapeDtypeStruct((pairs, _COUT), jnp.float32),
    )(hmax, hmin, s1p, s2p, gamma, beta)


def kernel(x, coords, W, b, gamma, beta):
    buf = _run_fps(coords)
    fps_t = jnp.transpose(buf.reshape(_M, _B, 3), (1, 0, 2))
    fps_coords = jnp.transpose(fps_t, (0, 2, 1))
    knn = _run_knn(fps_t, jnp.transpose(coords, (0, 2, 1)))
    zt = _run_mm(x, W, b)
    gidx = (knn + (jnp.arange(_B, dtype=jnp.int32) * _N)[:, None, None])
    gidx = gidx.reshape(_B * _M * _K)
    hmax, hmin, s1p, s2p = _gather_pool(zt.reshape(_B * _N, _COUT), gidx)
    y_t = _run_finalize(hmax, hmin, s1p, s2p, gamma, beta)
    y = jnp.transpose(y_t.reshape(_B, _M, _COUT), (0, 2, 1))
    return (y, fps_coords)
